# Initial kernel scaffold; baseline (speedup 1.0000x reference)
import jax
import jax.numpy as jnp
from jax.experimental import pallas as pl


def kernel(inputs, token_table, pos_table):
    raise NotImplementedError("write your pallas kernel here")


# speedup vs baseline: 5.0118x; 5.0118x over previous
import functools

import jax
import jax.numpy as jnp
from jax import lax
from jax.experimental import pallas as pl
from jax.experimental.pallas import tpu as pltpu
from jax.experimental.pallas import tpu_sc as plsc

_VOCAB = 100000
_EMBED_DIM = 32
_SEQ_LEN = 200
_BATCH = 4096
_MASK_VALUE = 1000.0

_NC, _NS, _L = 2, 16, 16
_NW = _NC * _NS
_HALF = _SEQ_LEN // 2
_HB_TOTAL = _BATCH * 2
_HB_PER_W = _HB_TOTAL // _NW
_G = 8
_GROUPS = _HB_PER_W // _G
_ROWS = _G * _HALF
_GVALS = _ROWS * _EMBED_DIM


def _sc_body(idx_hbm, table_hbm, pos_hbm, out_hbm,
             idx0, idx1, in0, in1, out0, out1, posb,
             gsem0, gsem1, wsem0, wsem1):
  wid = lax.axis_index("s") * _NC + lax.axis_index("c")
  idxb = (idx0, idx1)
  inb = (in0, in1)
  outb = (out0, out1)
  gsem = (gsem0, gsem1)
  wsem = (wsem0, wsem1)

  pltpu.sync_copy(pos_hbm, posb)

  def stage_group(t, b):
    hb_base = wid * _HB_PER_W + t * _G
    pltpu.sync_copy(idx_hbm.at[pl.ds(hb_base, _G), :], idxb[b])
    for j in range(_G):
      pltpu.make_async_copy(
          table_hbm.at[idxb[b].at[j]],
          inb[b].at[pl.ds(j * _HALF, _HALF)],
          gsem[b],
      ).start()

  def drain_gathers(b):
    for j in range(_G):
      pltpu.make_async_copy(
          table_hbm.at[idxb[b].at[j]],
          inb[b].at[pl.ds(j * _HALF, _HALF)],
          gsem[b],
      ).wait()

  def writeback(t, b):
    off = (wid * _HB_PER_W + t * _G) * _HALF * _EMBED_DIM
    pltpu.make_async_copy(outb[b], out_hbm.at[pl.ds(off, _GVALS)], wsem[b]).start()

  def drain_writeback(t, b):
    off = (wid * _HB_PER_W + t * _G) * _HALF * _EMBED_DIM
    pltpu.make_async_copy(outb[b], out_hbm.at[pl.ds(off, _GVALS)], wsem[b]).wait()

  def compute_group(b):
    src = inb[b]
    dst = outb[b]

    @pl.loop(0, _HALF)
    def _row(rr):
      base = rr * _EMBED_DIM
      pe0 = posb[pl.ds(base, _L)]
      pe1 = posb[pl.ds(base + _L, _L)]
      po0 = posb[pl.ds(_HALF * _EMBED_DIM + base, _L)]
      po1 = posb[pl.ds(_HALF * _EMBED_DIM + base + _L, _L)]
      for j in range(_G):
        row = j * _HALF + rr
        v0 = src[row, pl.ds(0, _L)]
        v1 = src[row, pl.ds(_L, _L)]
        keep = jnp.any(jnp.logical_or(v0 != _MASK_VALUE, v1 != _MASK_VALUE))
        p0 = pe0 if j % 2 == 0 else po0
        p1 = pe1 if j % 2 == 0 else po1
        o0 = jnp.where(keep, v0 + p0, p0)
        o1 = jnp.where(keep, v1 + p1, p1)
        off = j * _HALF * _EMBED_DIM + base
        dst[pl.ds(off, _L)] = o0
        dst[pl.ds(off + _L, _L)] = o1

  stage_group(0, 0)
  stage_group(1, 1)

  @pl.loop(0, _GROUPS // 2)
  def _outer(i):
    for b in range(2):
      t = 2 * i + b

      @pl.when(i >= 1)
      def _():
        drain_writeback(t - 2, b)

      drain_gathers(b)
      compute_group(b)
      writeback(t, b)

      @pl.when(i <= _GROUPS // 2 - 2)
      def _():
        stage_group(t + 2, b)

  drain_writeback(_GROUPS - 2, 0)
  drain_writeback(_GROUPS - 1, 1)


@jax.jit
def _board_embedding(idx, token_table, pos_flat):
  mesh = plsc.VectorSubcoreMesh(
      core_axis_name="c", subcore_axis_name="s",
      num_cores=_NC, num_subcores=_NS)
  return pl.kernel(
      _sc_body,
      out_type=jax.ShapeDtypeStruct((_BATCH * _SEQ_LEN * _EMBED_DIM,),
                                    jnp.float32),
      mesh=mesh,
      compiler_params=pltpu.CompilerParams(
          needs_layout_passes=False, use_tc_tiling_on_sc=False),
      scratch_types=[
          pltpu.VMEM((_G, _HALF), jnp.int32),
          pltpu.VMEM((_G, _HALF), jnp.int32),
          pltpu.VMEM((_ROWS, _EMBED_DIM), jnp.float32),
          pltpu.VMEM((_ROWS, _EMBED_DIM), jnp.float32),
          pltpu.VMEM((_GVALS,), jnp.float32),
          pltpu.VMEM((_GVALS,), jnp.float32),
          pltpu.VMEM((_SEQ_LEN * _EMBED_DIM,), jnp.float32),
          pltpu.SemaphoreType.DMA,
          pltpu.SemaphoreType.DMA,
          pltpu.SemaphoreType.DMA,
          pltpu.SemaphoreType.DMA,
      ],
  )(idx, token_table, pos_flat)


def kernel(inputs, token_table, pos_table):
  idx = inputs.reshape(_HB_TOTAL, _HALF)
  pos_flat = pos_table.reshape(-1)
  out = _board_embedding(idx, token_table, pos_flat)
  return out.reshape(_BATCH, _SEQ_LEN, _EMBED_DIM)
